# baseline (speedup 1.0000x reference)
import functools

import jax
import jax.numpy as jnp
from jax import lax
from jax.experimental import pallas as pl
from jax.experimental.pallas import tpu as pltpu
from jax.experimental.pallas import tpu_sc as plsc

B, P, E, H = 8, 32, 4096, 64
R = B * P
NC, NS, L = 2, 16, 16
NW = NC * NS
RPW = R // NW
NV = E // L

_CONSTS = {}


def _gumbel_weights():
    if "w" not in _CONSTS:
        g = jax.random.gumbel(jax.random.key(42), (B, P, E), jnp.float32)
        _CONSTS["w"] = jnp.exp(g).reshape(R, E)
    return _CONSTS["w"]


def _body(probs_hbm, w_hbm, idx_rows_hbm, emb_hbm,
          sel_out, prob_out, emb_out,
          pv0, wv0, pv1, wv1,
          gi_emb, gi_row, ind_rows, emb_rows, sel_stage, prob_stage,
          sem0, sem1, semg):
    wid = lax.axis_index("s") * NC + lax.axis_index("c")
    base = wid * RPW
    lane = lax.iota(jnp.int32, L)

    def shuffle(v, perm):
        return v.at[perm].get(mode="promise_in_bounds")

    def all_lanes_reduce(v, op):
        for sh in (1, 2, 4, 8):
            v = op(v, shuffle(v, lane ^ sh))
        return v

    bufs = ((pv0, wv0, sem0), (pv1, wv1, sem1))

    def start(j, slot):
        pv, wv, sem = bufs[slot]
        r = base + j
        return (pltpu.async_copy(probs_hbm.at[r], pv, sem),
                pltpu.async_copy(w_hbm.at[r], wv, sem))

    copies = {0: start(0, 0)}

    acc_sel = jnp.zeros((L,), jnp.int32)
    acc_row = jnp.zeros((L,), jnp.int32)
    acc_lane = jnp.zeros((L,), jnp.int32)
    acc_prob = jnp.zeros((L,), jnp.float32)

    for j in range(RPW):
        if j + 1 < RPW:
            copies[j + 1] = start(j + 1, (j + 1) % 2)
        for c in copies[j]:
            c.wait()
        pv, wv, _ = bufs[j % 2]
        r = base + j

        def sum_body(i, acc):
            return acc + pv[pl.ds(i * L, L)]
        acc = lax.fori_loop(0, NV, sum_body, jnp.zeros((L,), jnp.float32))
        inv = jnp.float32(1.0) / all_lanes_reduce(acc, jnp.add)

        def amax_body(i, carry):
            vmax, varg = carry
            t = (pv[pl.ds(i * L, L)] * inv + jnp.float32(1e-12)) * wv[pl.ds(i * L, L)]
            cond = t > vmax
            varg = jnp.where(cond, i * L + lane, varg)
            vmax = jnp.where(cond, t, vmax)
            return vmax, varg

        vmax, varg = lax.fori_loop(
            0, NV, amax_body,
            (jnp.full((L,), -1.0, jnp.float32), jnp.zeros((L,), jnp.int32)))
        m = all_lanes_reduce(vmax, jnp.maximum)
        cand = jnp.where(vmax == m, varg, jnp.int32(2**30))
        ev = all_lanes_reduce(cand, jnp.minimum)

        pe = plsc.load_gather(pv, [ev])
        lm = lane == j
        acc_prob = jnp.where(lm, pe * inv, acc_prob)
        acc_sel = jnp.where(lm, r * E + ev, acc_sel)
        acc_row = jnp.where(lm, r * (E // L) + (ev >> 4), acc_row)
        acc_lane = jnp.where(lm, ev & (L - 1), acc_lane)

    gi_emb[...] = acc_sel
    gi_row[...] = acc_row
    c1 = pltpu.async_copy(emb_hbm.at[gi_emb], emb_rows, semg)
    c2 = pltpu.async_copy(idx_rows_hbm.at[gi_row], ind_rows, semg)
    c1.wait()
    c2.wait()

    sel_stage[...] = plsc.load_gather(ind_rows, [lane, acc_lane])
    prob_stage[...] = acc_prob
    pltpu.sync_copy(sel_stage.at[pl.ds(0, RPW)], sel_out.at[pl.ds(base, RPW)])
    pltpu.sync_copy(prob_stage.at[pl.ds(0, RPW)], prob_out.at[pl.ds(base, RPW)])
    pltpu.sync_copy(emb_rows.at[pl.ds(0, RPW)], emb_out.at[pl.ds(base, RPW)])


@jax.jit
def _run(probs2d, w2d, idx_rows, emb_flat):
    mesh = plsc.VectorSubcoreMesh(core_axis_name="c", subcore_axis_name="s",
                                  num_cores=NC, num_subcores=NS)
    return pl.kernel(
        _body,
        out_type=[
            jax.ShapeDtypeStruct((R,), jnp.int32),
            jax.ShapeDtypeStruct((R,), jnp.float32),
            jax.ShapeDtypeStruct((R, H), jnp.float32),
        ],
        mesh=mesh,
        scratch_types=[
            pltpu.VMEM((E,), jnp.float32),
            pltpu.VMEM((E,), jnp.float32),
            pltpu.VMEM((E,), jnp.float32),
            pltpu.VMEM((E,), jnp.float32),
            pltpu.VMEM((L,), jnp.int32),
            pltpu.VMEM((L,), jnp.int32),
            pltpu.VMEM((L, L), jnp.int32),
            pltpu.VMEM((L, H), jnp.float32),
            pltpu.VMEM((L,), jnp.int32),
            pltpu.VMEM((L,), jnp.float32),
            pltpu.SemaphoreType.DMA,
            pltpu.SemaphoreType.DMA,
            pltpu.SemaphoreType.DMA,
        ],
        compiler_params=pltpu.CompilerParams(
            needs_layout_passes=False, use_tc_tiling_on_sc=False),
    )(probs2d, w2d, idx_rows, emb_flat)


def kernel(probs, embeddings, indices):
    sel, prob, emb = _run(
        probs.reshape(R, E),
        _gumbel_weights(),
        indices.reshape(R * E // L, L),
        embeddings.reshape(R * E, H),
    )
    return sel.reshape(B, P), prob.reshape(B, P), emb.reshape(B, P, H)


# speedup vs baseline: 12.5796x; 12.5796x over previous
import jax
import jax.numpy as jnp
from jax import lax
from jax.experimental import pallas as pl
from jax.experimental.pallas import tpu as pltpu
from jax.experimental.pallas import tpu_sc as plsc

B, P, E, H = 8, 32, 4096, 64
R = B * P
NC, NS, L = 2, 16, 16
NW = NC * NS
RPW = R // NW
NV = E // L

_CONSTS = {}


def _gumbel_weights():
    if "w" not in _CONSTS:
        g = jax.random.gumbel(jax.random.key(42), (B, P, E), jnp.float32)
        _CONSTS["w"] = jnp.exp(g).reshape(R, E)
    return _CONSTS["w"]


def _body(probs_hbm, w_hbm, ind_hbm, emb_hbm,
          sel_out, prob_out, emb_out,
          pv, wv, emb_g0, emb_g1, ind_g, emb_stage, sel_stage, prob_stage,
          sem_s, sem_g, sem_i):
    wid = lax.axis_index("s") * NC + lax.axis_index("c")
    base = wid * RPW
    b = base // P
    p0 = base % P
    lane = lax.iota(jnp.int32, L)

    cp = pltpu.async_copy(probs_hbm.at[pl.ds(base, RPW)], pv, sem_s)
    cw = pltpu.async_copy(w_hbm.at[pl.ds(base, RPW)], wv, sem_s)
    cp.wait()
    cw.wait()

    emb_bufs = (emb_g0, emb_g1)
    emb_copies = {}
    ind_copies = []
    els = []
    acc_prob = jnp.zeros((L,), jnp.float32)

    def extract_emb(j):
        emb_copies[j].wait()
        g = emb_bufs[j % 2]
        elv = jnp.full((L,), els[j], jnp.int32)
        for k in range(H // L):
            col = plsc.load_gather(g, [k * L + lane, elv])
            emb_stage[j, pl.ds(k * L, L)] = col

    for j in range(RPW):
        def sum_body(i, acc):
            return acc + pv[j, pl.ds(i * L, L)]
        acc = lax.fori_loop(0, NV, sum_body, jnp.zeros((L,), jnp.float32),
                            unroll=8)
        s = jnp.sum(acc)
        cvec = jnp.full((L,), s * jnp.float32(1e-12))

        def amax_body(i, carry):
            vmax, varg = carry
            t = (pv[j, pl.ds(i * L, L)] + cvec) * wv[j, pl.ds(i * L, L)]
            cond = t > vmax
            varg = jnp.where(cond, i * L + lane, varg)
            vmax = jnp.where(cond, t, vmax)
            return vmax, varg

        vmax, varg = lax.fori_loop(
            0, NV, amax_body,
            (jnp.full((L,), -1.0, jnp.float32), jnp.zeros((L,), jnp.int32)),
            unroll=4)
        m = jnp.max(vmax)
        e = jnp.min(jnp.where(vmax == m, varg, jnp.int32(2**30)))
        eg = e >> 7
        els.append(e & 127)

        pe = plsc.load_gather(pv, [jnp.full((L,), j, jnp.int32),
                                   jnp.full((L,), e, jnp.int32)])
        acc_prob = jnp.where(lane == j, pe / jnp.full((L,), s), acc_prob)

        emb_copies[j] = pltpu.async_copy(
            emb_hbm.at[b, p0 + j, slice(None), pl.ds(eg * 128, 128)],
            emb_bufs[j % 2], sem_g)
        ind_copies.append(pltpu.async_copy(
            ind_hbm.at[base + j, pl.ds(eg * 128, 128)], ind_g.at[j], sem_i))
        if j > 0:
            extract_emb(j - 1)
    extract_emb(RPW - 1)

    for c in ind_copies:
        c.wait()
    acc_sel = jnp.zeros((L,), jnp.int32)
    for j in range(RPW):
        iv = plsc.load_gather(ind_g, [jnp.full((L,), j, jnp.int32),
                                      jnp.full((L,), els[j], jnp.int32)])
        acc_sel = jnp.where(lane == j, iv, acc_sel)

    sel_stage[...] = acc_sel
    prob_stage[...] = acc_prob
    pltpu.sync_copy(sel_stage.at[pl.ds(0, RPW)], sel_out.at[pl.ds(base, RPW)])
    pltpu.sync_copy(prob_stage.at[pl.ds(0, RPW)], prob_out.at[pl.ds(base, RPW)])
    pltpu.sync_copy(emb_stage, emb_out.at[pl.ds(base, RPW)])


@jax.jit
def _run(probs2d, w2d, ind2d, emb_t):
    mesh = plsc.VectorSubcoreMesh(core_axis_name="c", subcore_axis_name="s",
                                  num_cores=NC, num_subcores=NS)
    return pl.kernel(
        _body,
        out_type=[
            jax.ShapeDtypeStruct((R,), jnp.int32),
            jax.ShapeDtypeStruct((R,), jnp.float32),
            jax.ShapeDtypeStruct((R, H), jnp.float32),
        ],
        mesh=mesh,
        scratch_types=[
            pltpu.VMEM((RPW, E), jnp.float32),
            pltpu.VMEM((RPW, E), jnp.float32),
            pltpu.VMEM((H, 128), jnp.float32),
            pltpu.VMEM((H, 128), jnp.float32),
            pltpu.VMEM((RPW, 128), jnp.int32),
            pltpu.VMEM((RPW, H), jnp.float32),
            pltpu.VMEM((L,), jnp.int32),
            pltpu.VMEM((L,), jnp.float32),
            pltpu.SemaphoreType.DMA,
            pltpu.SemaphoreType.DMA,
            pltpu.SemaphoreType.DMA,
        ],
        compiler_params=pltpu.CompilerParams(
            needs_layout_passes=False, use_tc_tiling_on_sc=True),
    )(probs2d, w2d, ind2d, emb_t)


def kernel(probs, embeddings, indices):
    sel, prob, emb = _run(
        probs.reshape(R, E),
        _gumbel_weights(),
        indices.reshape(R, E),
        jnp.swapaxes(embeddings, 2, 3),
    )
    return sel.reshape(B, P), prob.reshape(B, P), emb.reshape(B, P, H)
